# Initial kernel scaffold; baseline (speedup 1.0000x reference)
import jax
import jax.numpy as jnp
from jax.experimental import pallas as pl


def kernel(attribute, table):
    raise NotImplementedError("write your pallas kernel here")


# speedup vs baseline: 1.6948x; 1.6948x over previous
import functools

import jax
import jax.numpy as jnp
from jax import lax
from jax.experimental import pallas as pl
from jax.experimental.pallas import tpu as pltpu
from jax.experimental.pallas import tpu_sc as plsc

D_MODEL = 64
ROWS_PER_GATHER = 128


def _make_sc_gather(n_idx_rows, steps_per_tile, num_cores):
    mesh = plsc.VectorSubcoreMesh(core_axis_name="c", subcore_axis_name="s")

    @functools.partial(
        pl.kernel,
        mesh=mesh,
        out_type=jax.ShapeDtypeStruct(
            (n_idx_rows * ROWS_PER_GATHER, D_MODEL), jnp.float32
        ),
        scratch_types=[
            pltpu.VMEM((steps_per_tile, ROWS_PER_GATHER), jnp.int32),
            pltpu.VMEM((ROWS_PER_GATHER, D_MODEL), jnp.float32),
            pltpu.SemaphoreType.DMA,
        ],
        compiler_params=pltpu.CompilerParams(use_tc_tiling_on_sc=False),
    )
    def k(idx_hbm, table_hbm, out_hbm, idx_v, rows_v, sem):
        wid = lax.axis_index("s") * num_cores + lax.axis_index("c")
        row0 = wid * steps_per_tile
        pltpu.sync_copy(idx_hbm.at[pl.ds(row0, steps_per_tile)], idx_v)

        def step(t, carry):
            pltpu.async_copy(table_hbm.at[idx_v.at[t]], rows_v, sem).wait()
            pltpu.sync_copy(
                rows_v,
                out_hbm.at[pl.ds((row0 + t) * ROWS_PER_GATHER, ROWS_PER_GATHER)],
            )
            return carry

        lax.fori_loop(0, steps_per_tile, step, 0)

    return k


def kernel(attribute, table):
    b0, b1 = attribute.shape
    n = b0 * b1
    info = plsc.get_sparse_core_info()
    nw = info.num_cores * info.num_subcores
    n_idx_rows = n // ROWS_PER_GATHER
    steps_per_tile = n_idx_rows // nw
    idx = attribute.reshape(n_idx_rows, ROWS_PER_GATHER).astype(jnp.int32)
    out = _make_sc_gather(n_idx_rows, steps_per_tile, info.num_cores)(idx, table)
    return out.reshape(b0, b1, D_MODEL)


# speedup vs baseline: 1.8719x; 1.1045x over previous
import functools

import jax
import jax.numpy as jnp
from jax import lax
from jax.experimental import pallas as pl
from jax.experimental.pallas import tpu as pltpu
from jax.experimental.pallas import tpu_sc as plsc

D_MODEL = 64
ROWS_PER_GATHER = 128
NBUF = 8


def _make_sc_gather(n_idx_rows, steps_per_tile, num_cores):
    mesh = plsc.VectorSubcoreMesh(core_axis_name="c", subcore_axis_name="s")
    n_groups = steps_per_tile // NBUF

    @functools.partial(
        pl.kernel,
        mesh=mesh,
        out_type=jax.ShapeDtypeStruct(
            (n_idx_rows * ROWS_PER_GATHER, D_MODEL), jnp.float32
        ),
        scratch_types=[
            pltpu.VMEM((steps_per_tile, ROWS_PER_GATHER), jnp.int32),
            pltpu.VMEM((NBUF, ROWS_PER_GATHER, D_MODEL), jnp.float32),
            pltpu.SemaphoreType.DMA((NBUF,)),
            pltpu.SemaphoreType.DMA((NBUF,)),
        ],
        compiler_params=pltpu.CompilerParams(use_tc_tiling_on_sc=False),
    )
    def k(idx_hbm, table_hbm, out_hbm, idx_v, rows_v, gsem, osem):
        wid = lax.axis_index("s") * num_cores + lax.axis_index("c")
        row0 = wid * steps_per_tile
        pltpu.sync_copy(idx_hbm.at[pl.ds(row0, steps_per_tile)], idx_v)

        def gather(t, b):
            pltpu.async_copy(table_hbm.at[idx_v.at[t]], rows_v.at[b], gsem.at[b])

        def wait_gather(b):
            pltpu.make_async_copy(
                table_hbm.at[pl.ds(0, ROWS_PER_GATHER)], rows_v.at[b], gsem.at[b]
            ).wait()

        def writeback(t, b):
            pltpu.async_copy(
                rows_v.at[b],
                out_hbm.at[pl.ds((row0 + t) * ROWS_PER_GATHER, ROWS_PER_GATHER)],
                osem.at[b],
            )

        def wait_writeback(b):
            pltpu.make_async_copy(
                rows_v.at[b],
                out_hbm.at[pl.ds(0, ROWS_PER_GATHER)],
                osem.at[b],
            ).wait()

        for b in range(NBUF):
            gather(b, b)

        def group(gi, carry):
            t0 = gi * NBUF
            for b in range(NBUF):
                wait_gather(b)
                writeback(t0 + b, b)
            for b in range(NBUF):
                wait_writeback(b)
                gather(t0 + b + NBUF, b)
            return carry

        lax.fori_loop(0, n_groups - 1, group, 0)

        t0 = (n_groups - 1) * NBUF
        for b in range(NBUF):
            wait_gather(b)
            writeback(t0 + b, b)
        for b in range(NBUF):
            wait_writeback(b)

    return k


def kernel(attribute, table):
    b0, b1 = attribute.shape
    n = b0 * b1
    info = plsc.get_sparse_core_info()
    nw = info.num_cores * info.num_subcores
    n_idx_rows = n // ROWS_PER_GATHER
    steps_per_tile = n_idx_rows // nw
    idx = attribute.reshape(n_idx_rows, ROWS_PER_GATHER).astype(jnp.int32)
    out = _make_sc_gather(n_idx_rows, steps_per_tile, info.num_cores)(idx, table)
    return out.reshape(b0, b1, D_MODEL)
